# baseline (speedup 1.0000x reference)
import functools

import jax
import jax.numpy as jnp
from jax import lax
from jax.experimental import pallas as pl
from jax.experimental.pallas import tpu as pltpu
from jax.experimental.pallas import tpu_sc as plsc

N_NODES = 10000
N_EDGES = 320000
D_IN = 128
D_HID = 18
D_OUT = 7

DP1 = 24
DP2 = 8
NP = 10240

NC = 2
NS = 16
NW = NC * NS
EDGES_PER_W = N_EDGES // NW
CHUNK = 80
EPW_PAD = 10400
NCHUNK = EPW_PAD // CHUNK
NBUF = 10
ROWS_PER_TILE = NP // NS


def _mm_body(x_ref, w_ref, o_ref):
    o_ref[...] = jnp.dot(x_ref[...], w_ref[...],
                         preferred_element_type=jnp.float32)


def _tc_matmul(x, w, bm, m_out):
    m, k = x.shape
    n = w.shape[1]
    return pl.pallas_call(
        _mm_body,
        grid=(m // bm,),
        in_specs=[pl.BlockSpec((bm, k), lambda i: (i, 0)),
                  pl.BlockSpec((k, n), lambda i: (0, 0))],
        out_specs=pl.BlockSpec((bm, n), lambda i: (i, 0)),
        out_shape=jax.ShapeDtypeStruct((m_out, n), jnp.float32),
    )(x, w)


def _fused_body(p0_ref, p1_ref, b_ref, w_ref, o_ref):
    h = jnp.maximum(p0_ref[...] + p1_ref[...] + b_ref[...], 0.0)
    o_ref[...] = jnp.dot(h, w_ref[...], preferred_element_type=jnp.float32)


def _tc_relu_matmul(p0, p1, b, w, bm):
    m, k = p0.shape
    n = w.shape[1]
    return pl.pallas_call(
        _fused_body,
        grid=(m // bm,),
        in_specs=[pl.BlockSpec((bm, k), lambda i: (i, 0)),
                  pl.BlockSpec((bm, k), lambda i: (i, 0)),
                  pl.BlockSpec((1, k), lambda i: (0, 0)),
                  pl.BlockSpec((k, n), lambda i: (0, 0))],
        out_specs=pl.BlockSpec((bm, n), lambda i: (i, 0)),
        out_shape=jax.ShapeDtypeStruct((m, n), jnp.float32),
    )(p0, p1, b, w)


def _final_body(p0_ref, p1_ref, b_ref, o_ref):
    s = p0_ref[...] + p1_ref[...] + b_ref[...]
    o_ref[...] = s[:, :D_OUT]


def _tc_final(p0, p1, b, bm):
    m, k = p0.shape
    return pl.pallas_call(
        _final_body,
        grid=(m // bm,),
        in_specs=[pl.BlockSpec((bm, k), lambda i: (i, 0)),
                  pl.BlockSpec((bm, k), lambda i: (i, 0)),
                  pl.BlockSpec((1, k), lambda i: (0, 0))],
        out_specs=pl.BlockSpec((bm, D_OUT), lambda i: (i, 0)),
        out_shape=jax.ShapeDtypeStruct((m, D_OUT), jnp.float32),
    )(p0, p1, b)


def _sc_aggregate(support, src2d, dst2d, zeros_hbm, dp):
    mesh = plsc.VectorSubcoreMesh(core_axis_name="c", subcore_axis_name="s")

    @functools.partial(
        pl.kernel,
        out_type=jax.ShapeDtypeStruct((NC, NP, dp), jnp.float32),
        mesh=mesh,
        compiler_params=pltpu.CompilerParams(use_tc_tiling_on_sc=False),
        scratch_types=[
            pltpu.VMEM((NCHUNK, CHUNK), jnp.int32),
            pltpu.VMEM((NCHUNK, CHUNK), jnp.int32),
            pltpu.VMEM((NBUF, CHUNK, dp), jnp.float32),
            pltpu.VMEM_SHARED((NP, dp), jnp.float32),
            pltpu.VMEM_SHARED((NP, dp), jnp.float32),
            pltpu.SemaphoreType.DMA((NBUF,)),
            pltpu.SemaphoreType.DMA((NBUF,)),
        ],
    )
    def k(sup_hbm, src_hbm, dst_hbm, zero_hbm, out_hbm,
          src_v, dst_v, rows_v, agg_sh, sup_sh, gsem, ssem):
        cid = lax.axis_index("c")
        sid = lax.axis_index("s")
        wid = sid * NC + cid

        pltpu.sync_copy(
            zero_hbm.at[pl.ds(sid * ROWS_PER_TILE, ROWS_PER_TILE)],
            agg_sh.at[pl.ds(sid * ROWS_PER_TILE, ROWS_PER_TILE)],
        )

        pltpu.sync_copy(src_hbm.at[wid], src_v)
        pltpu.sync_copy(dst_hbm.at[wid], dst_v)

        pltpu.sync_copy(
            sup_hbm.at[pl.ds(sid * ROWS_PER_TILE, ROWS_PER_TILE)],
            sup_sh.at[pl.ds(sid * ROWS_PER_TILE, ROWS_PER_TILE)],
        )

        plsc.subcore_barrier()

        for b in range(NBUF):
            pltpu.async_copy(sup_sh.at[src_v.at[b]], rows_v.at[b], gsem.at[b])

        @pl.loop(0, NCHUNK, step=NBUF)
        def _(kk):
            for b in range(NBUF):
                pltpu.make_async_copy(
                    sup_sh.at[src_v.at[kk + b]], rows_v.at[b], gsem.at[b]
                ).wait()
                pltpu.async_copy(
                    rows_v.at[b], agg_sh.at[dst_v.at[kk + b]], ssem.at[b],
                    add=True,
                )
            for b in range(NBUF):
                pltpu.make_async_copy(
                    rows_v.at[b], agg_sh.at[dst_v.at[kk + b]], ssem.at[b]
                ).wait()
                nxt = kk + NBUF + b

                @pl.when(nxt < NCHUNK)
                def _():
                    pltpu.async_copy(
                        sup_sh.at[src_v.at[nxt]], rows_v.at[b], gsem.at[b]
                    )

        plsc.subcore_barrier()

        pltpu.sync_copy(
            agg_sh.at[pl.ds(sid * ROWS_PER_TILE, ROWS_PER_TILE)],
            out_hbm.at[cid, pl.ds(sid * ROWS_PER_TILE, ROWS_PER_TILE)],
        )

    return k(support, src2d, dst2d, zeros_hbm)


def kernel(adjacency, feature_matrix, W1, b1, W2, b2):
    npad = EPW_PAD - EDGES_PER_W
    src2d = jnp.concatenate(
        [adjacency[0].reshape(NW, EDGES_PER_W),
         jnp.zeros((NW, npad), jnp.int32)], axis=1
    ).reshape(NW, NCHUNK, CHUNK)
    dst2d = jnp.concatenate(
        [adjacency[1].reshape(NW, EDGES_PER_W),
         jnp.full((NW, npad), N_NODES, jnp.int32)], axis=1
    ).reshape(NW, NCHUNK, CHUNK)

    W1p = jnp.zeros((D_IN, DP1), jnp.float32).at[:, :D_HID].set(W1)
    b1p = jnp.zeros((1, DP1), jnp.float32).at[0, :D_HID].set(b1)
    W2p = jnp.zeros((DP1, DP2), jnp.float32).at[:D_HID, :D_OUT].set(W2)
    b2p = jnp.zeros((1, DP2), jnp.float32).at[0, :D_OUT].set(b2)

    z1 = jnp.zeros((NP, DP1), jnp.float32)
    z2 = jnp.zeros((NP, DP2), jnp.float32)
    support1 = _tc_matmul(feature_matrix, W1p, bm=1000, m_out=NP)
    part1 = _sc_aggregate(support1, src2d, dst2d, z1, DP1)
    support2 = _tc_relu_matmul(part1[0], part1[1], b1p, W2p, bm=1024)
    part2 = _sc_aggregate(support2, src2d, dst2d, z2, DP2)
    return _tc_final(part2[0], part2[1], b2p, bm=1024)[:N_NODES]


# speedup vs baseline: 1.0858x; 1.0858x over previous
import functools

import jax
import jax.numpy as jnp
from jax import lax
from jax.experimental import pallas as pl
from jax.experimental.pallas import tpu as pltpu
from jax.experimental.pallas import tpu_sc as plsc

N_NODES = 10000
N_EDGES = 320000
D_IN = 128
D_HID = 18
D_OUT = 7

DP1 = 24
DP2 = 8
NP = 10240

NC = 2
NS = 16
NW = NC * NS
EDGES_PER_W = N_EDGES // NW
CHUNK = 80
NCHUNK = EDGES_PER_W // CHUNK
NBUF = 5
ROWS_PER_TILE = NP // NS


def _mm_body(x_ref, w_ref, o_ref):
    o_ref[...] = jnp.dot(x_ref[...], w_ref[...],
                         preferred_element_type=jnp.float32)


def _tc_matmul(x, w, bm, m_out):
    m, k = x.shape
    n = w.shape[1]
    return pl.pallas_call(
        _mm_body,
        grid=(m // bm,),
        in_specs=[pl.BlockSpec((bm, k), lambda i: (i, 0)),
                  pl.BlockSpec((k, n), lambda i: (0, 0))],
        out_specs=pl.BlockSpec((bm, n), lambda i: (i, 0)),
        out_shape=jax.ShapeDtypeStruct((m_out, n), jnp.float32),
    )(x, w)


def _fused_body(p0_ref, p1_ref, b_ref, w_ref, o_ref):
    h = jnp.maximum(p0_ref[...] + p1_ref[...] + b_ref[...], 0.0)
    o_ref[...] = jnp.dot(h, w_ref[...], preferred_element_type=jnp.float32)


def _tc_relu_matmul(p0, p1, b, w, bm):
    m, k = p0.shape
    n = w.shape[1]
    return pl.pallas_call(
        _fused_body,
        grid=(m // bm,),
        in_specs=[pl.BlockSpec((bm, k), lambda i: (i, 0)),
                  pl.BlockSpec((bm, k), lambda i: (i, 0)),
                  pl.BlockSpec((1, k), lambda i: (0, 0)),
                  pl.BlockSpec((k, n), lambda i: (0, 0))],
        out_specs=pl.BlockSpec((bm, n), lambda i: (i, 0)),
        out_shape=jax.ShapeDtypeStruct((m, n), jnp.float32),
    )(p0, p1, b, w)


def _final_body(p0_ref, p1_ref, b_ref, o_ref):
    s = p0_ref[...] + p1_ref[...] + b_ref[...]
    o_ref[...] = s[:, :D_OUT]


def _tc_final(p0, p1, b, bm):
    m, k = p0.shape
    return pl.pallas_call(
        _final_body,
        grid=(m // bm,),
        in_specs=[pl.BlockSpec((bm, k), lambda i: (i, 0)),
                  pl.BlockSpec((bm, k), lambda i: (i, 0)),
                  pl.BlockSpec((1, k), lambda i: (0, 0))],
        out_specs=pl.BlockSpec((bm, D_OUT), lambda i: (i, 0)),
        out_shape=jax.ShapeDtypeStruct((m, D_OUT), jnp.float32),
    )(p0, p1, b)


def _sc_aggregate(support, src2d, dst2d, zeros_hbm, dp):
    mesh = plsc.VectorSubcoreMesh(core_axis_name="c", subcore_axis_name="s")

    @functools.partial(
        pl.kernel,
        out_type=jax.ShapeDtypeStruct((NC, NP, dp), jnp.float32),
        mesh=mesh,
        compiler_params=pltpu.CompilerParams(use_tc_tiling_on_sc=False),
        scratch_types=[
            pltpu.VMEM((NCHUNK, CHUNK), jnp.int32),
            pltpu.VMEM((NCHUNK, CHUNK), jnp.int32),
            pltpu.VMEM((NBUF, CHUNK, dp), jnp.float32),
            pltpu.VMEM_SHARED((NP, dp), jnp.float32),
            pltpu.VMEM_SHARED((NP, dp), jnp.float32),
            pltpu.SemaphoreType.DMA((NBUF,)),
            pltpu.SemaphoreType.DMA((NBUF,)),
        ],
    )
    def k(sup_hbm, src_hbm, dst_hbm, zero_hbm, out_hbm,
          src_v, dst_v, rows_v, agg_sh, sup_sh, gsem, ssem):
        cid = lax.axis_index("c")
        sid = lax.axis_index("s")
        wid = sid * NC + cid

        pltpu.sync_copy(
            zero_hbm.at[pl.ds(sid * ROWS_PER_TILE, ROWS_PER_TILE)],
            agg_sh.at[pl.ds(sid * ROWS_PER_TILE, ROWS_PER_TILE)],
        )

        pltpu.sync_copy(src_hbm.at[wid], src_v)
        pltpu.sync_copy(dst_hbm.at[wid], dst_v)

        pltpu.sync_copy(
            sup_hbm.at[pl.ds(sid * ROWS_PER_TILE, ROWS_PER_TILE)],
            sup_sh.at[pl.ds(sid * ROWS_PER_TILE, ROWS_PER_TILE)],
        )

        plsc.subcore_barrier()

        for b in range(NBUF):
            pltpu.async_copy(sup_sh.at[src_v.at[b]], rows_v.at[b], gsem.at[b])

        @pl.loop(0, NCHUNK, step=NBUF)
        def _(kk):
            for b in range(NBUF):
                pltpu.make_async_copy(
                    sup_sh.at[src_v.at[kk + b]], rows_v.at[b], gsem.at[b]
                ).wait()
                pltpu.async_copy(
                    rows_v.at[b], agg_sh.at[dst_v.at[kk + b]], ssem.at[b],
                    add=True,
                )
            for b in range(NBUF):
                pltpu.make_async_copy(
                    rows_v.at[b], agg_sh.at[dst_v.at[kk + b]], ssem.at[b]
                ).wait()
                nxt = kk + NBUF + b

                @pl.when(nxt < NCHUNK)
                def _():
                    pltpu.async_copy(
                        sup_sh.at[src_v.at[nxt]], rows_v.at[b], gsem.at[b]
                    )

        plsc.subcore_barrier()

        pltpu.sync_copy(
            agg_sh.at[pl.ds(sid * ROWS_PER_TILE, ROWS_PER_TILE)],
            out_hbm.at[cid, pl.ds(sid * ROWS_PER_TILE, ROWS_PER_TILE)],
        )

    return k(support, src2d, dst2d, zeros_hbm)


def kernel(adjacency, feature_matrix, W1, b1, W2, b2):
    src2d = adjacency[0].reshape(NW, NCHUNK, CHUNK)
    dst2d = adjacency[1].reshape(NW, NCHUNK, CHUNK)

    W1p = jnp.zeros((D_IN, DP1), jnp.float32).at[:, :D_HID].set(W1)
    b1p = jnp.zeros((1, DP1), jnp.float32).at[0, :D_HID].set(b1)
    W2p = jnp.zeros((DP1, DP2), jnp.float32).at[:D_HID, :D_OUT].set(W2)
    b2p = jnp.zeros((1, DP2), jnp.float32).at[0, :D_OUT].set(b2)

    z1 = jnp.zeros((NP, DP1), jnp.float32)
    z2 = jnp.zeros((NP, DP2), jnp.float32)
    support1 = _tc_matmul(feature_matrix, W1p, bm=1000, m_out=NP)
    part1 = _sc_aggregate(support1, src2d, dst2d, z1, DP1)
    support2 = _tc_relu_matmul(part1[0], part1[1], b1p, W2p, bm=1024)
    part2 = _sc_aggregate(support2, src2d, dst2d, z2, DP2)
    return _tc_final(part2[0], part2[1], b2p, bm=1024)[:N_NODES]


# speedup vs baseline: 1.1025x; 1.0153x over previous
import functools

import jax
import jax.numpy as jnp
from jax import lax
from jax.experimental import pallas as pl
from jax.experimental.pallas import tpu as pltpu
from jax.experimental.pallas import tpu_sc as plsc

N_NODES = 10000
N_EDGES = 320000
D_IN = 128
D_HID = 18
D_OUT = 7

DP1 = 24
DP2 = 8
NP = 10240

NC = 2
NS = 16
NW = NC * NS
EDGES_PER_W = N_EDGES // NW
CHUNK = 80
NCHUNK = EDGES_PER_W // CHUNK
NBUF = 5
ROWS_PER_TILE = NP // NS


def _mm_body(x_ref, w_ref, o_ref):
    o_ref[...] = jnp.dot(x_ref[...], w_ref[...],
                         preferred_element_type=jnp.float32)


def _tc_matmul(x, w, bm, m_out):
    m, k = x.shape
    n = w.shape[1]
    return pl.pallas_call(
        _mm_body,
        grid=(m // bm,),
        in_specs=[pl.BlockSpec((bm, k), lambda i: (i, 0)),
                  pl.BlockSpec((k, n), lambda i: (0, 0))],
        out_specs=pl.BlockSpec((bm, n), lambda i: (i, 0)),
        out_shape=jax.ShapeDtypeStruct((m_out, n), jnp.float32),
    )(x, w)


def _fused_body(p0_ref, p1_ref, b_ref, w_ref, o_ref):
    h = jnp.maximum(p0_ref[...] + p1_ref[...] + b_ref[...], 0.0)
    o_ref[...] = jnp.dot(h, w_ref[...], preferred_element_type=jnp.float32)


def _tc_relu_matmul(p0, p1, b, w, bm):
    m, k = p0.shape
    n = w.shape[1]
    return pl.pallas_call(
        _fused_body,
        grid=(m // bm,),
        in_specs=[pl.BlockSpec((bm, k), lambda i: (i, 0)),
                  pl.BlockSpec((bm, k), lambda i: (i, 0)),
                  pl.BlockSpec((1, k), lambda i: (0, 0)),
                  pl.BlockSpec((k, n), lambda i: (0, 0))],
        out_specs=pl.BlockSpec((bm, n), lambda i: (i, 0)),
        out_shape=jax.ShapeDtypeStruct((m, n), jnp.float32),
    )(p0, p1, b, w)


def _final_body(p0_ref, p1_ref, b_ref, o_ref):
    s = p0_ref[...] + p1_ref[...] + b_ref[...]
    o_ref[...] = s[:, :D_OUT]


def _tc_final(p0, p1, b, bm):
    m, k = p0.shape
    return pl.pallas_call(
        _final_body,
        grid=(m // bm,),
        in_specs=[pl.BlockSpec((bm, k), lambda i: (i, 0)),
                  pl.BlockSpec((bm, k), lambda i: (i, 0)),
                  pl.BlockSpec((1, k), lambda i: (0, 0))],
        out_specs=pl.BlockSpec((bm, D_OUT), lambda i: (i, 0)),
        out_shape=jax.ShapeDtypeStruct((m, D_OUT), jnp.float32),
    )(p0, p1, b)


def _sc_aggregate(support, src2d, dst2d, zeros_hbm, dp, stage_spmem):
    mesh = plsc.VectorSubcoreMesh(core_axis_name="c", subcore_axis_name="s")

    @functools.partial(
        pl.kernel,
        out_type=jax.ShapeDtypeStruct((NC, NP, dp), jnp.float32),
        mesh=mesh,
        compiler_params=pltpu.CompilerParams(use_tc_tiling_on_sc=False),
        scratch_types=[
            pltpu.VMEM((NCHUNK, CHUNK), jnp.int32),
            pltpu.VMEM((NCHUNK, CHUNK), jnp.int32),
            pltpu.VMEM((NBUF, CHUNK, dp), jnp.float32),
            pltpu.VMEM_SHARED((NP, dp), jnp.float32),
            pltpu.VMEM_SHARED((NP, dp), jnp.float32),
            pltpu.SemaphoreType.DMA((NBUF,)),
            pltpu.SemaphoreType.DMA((NBUF,)),
        ],
    )
    def k(sup_hbm, src_hbm, dst_hbm, zero_hbm, out_hbm,
          src_v, dst_v, rows_v, agg_sh, sup_sh, gsem, ssem):
        cid = lax.axis_index("c")
        sid = lax.axis_index("s")
        wid = sid * NC + cid

        pltpu.sync_copy(
            zero_hbm.at[pl.ds(sid * ROWS_PER_TILE, ROWS_PER_TILE)],
            agg_sh.at[pl.ds(sid * ROWS_PER_TILE, ROWS_PER_TILE)],
        )

        pltpu.sync_copy(src_hbm.at[wid], src_v)
        pltpu.sync_copy(dst_hbm.at[wid], dst_v)

        if stage_spmem:
            pltpu.sync_copy(
                sup_hbm.at[pl.ds(sid * ROWS_PER_TILE, ROWS_PER_TILE)],
                sup_sh.at[pl.ds(sid * ROWS_PER_TILE, ROWS_PER_TILE)],
            )
        sup_src = sup_sh if stage_spmem else sup_hbm

        plsc.subcore_barrier()

        for b in range(NBUF):
            pltpu.async_copy(sup_src.at[src_v.at[b]], rows_v.at[b], gsem.at[b])

        @pl.loop(0, NCHUNK, step=NBUF)
        def _(kk):
            for b in range(NBUF):
                pltpu.make_async_copy(
                    sup_src.at[src_v.at[kk + b]], rows_v.at[b], gsem.at[b]
                ).wait()
                pltpu.async_copy(
                    rows_v.at[b], agg_sh.at[dst_v.at[kk + b]], ssem.at[b],
                    add=True,
                )
            for b in range(NBUF):
                pltpu.make_async_copy(
                    rows_v.at[b], agg_sh.at[dst_v.at[kk + b]], ssem.at[b]
                ).wait()
                nxt = kk + NBUF + b

                @pl.when(nxt < NCHUNK)
                def _():
                    pltpu.async_copy(
                        sup_src.at[src_v.at[nxt]], rows_v.at[b], gsem.at[b]
                    )

        plsc.subcore_barrier()

        pltpu.sync_copy(
            agg_sh.at[pl.ds(sid * ROWS_PER_TILE, ROWS_PER_TILE)],
            out_hbm.at[cid, pl.ds(sid * ROWS_PER_TILE, ROWS_PER_TILE)],
        )

    return k(support, src2d, dst2d, zeros_hbm)


def kernel(adjacency, feature_matrix, W1, b1, W2, b2):
    src2d = adjacency[0].reshape(NW, NCHUNK, CHUNK)
    dst2d = adjacency[1].reshape(NW, NCHUNK, CHUNK)

    W1p = jnp.zeros((D_IN, DP1), jnp.float32).at[:, :D_HID].set(W1)
    b1p = jnp.zeros((1, DP1), jnp.float32).at[0, :D_HID].set(b1)
    W2p = jnp.zeros((DP1, DP2), jnp.float32).at[:D_HID, :D_OUT].set(W2)
    b2p = jnp.zeros((1, DP2), jnp.float32).at[0, :D_OUT].set(b2)

    z1 = jnp.zeros((NP, DP1), jnp.float32)
    z2 = jnp.zeros((NP, DP2), jnp.float32)
    support1 = _tc_matmul(feature_matrix, W1p, bm=1000, m_out=NP)
    part1 = _sc_aggregate(support1, src2d, dst2d, z1, DP1, stage_spmem=False)
    support2 = _tc_relu_matmul(part1[0], part1[1], b1p, W2p, bm=1024)
    part2 = _sc_aggregate(support2, src2d, dst2d, z2, DP2, stage_spmem=True)
    return _tc_final(part2[0], part2[1], b2p, bm=1024)[:N_NODES]
